# baseline (speedup 1.0000x reference)
import jax
import jax.numpy as jnp
from jax.experimental import pallas as pl
from jax.experimental.pallas import tpu as pltpu

_B, _S = 1, 128
_HID = 768
_INTER = 768
_NEXP = 16
_LIMIT = 7.0
_EPS = 1e-05
_ALPHA = 1.702


def _moe_kernel(x_ref, nw_ref, gw_ref, gb_ref,
                w1_ref, b1_ref, w2_ref, b2_ref,
                out_ref,
                normedT_ref, c_ref, b1t_ref, acc_ref, h_ref):
    g = pl.program_id(0)

    @pl.when(g == 0)
    def _prologue():
        xf = x_ref[0].astype(jnp.bfloat16).astype(jnp.float32)
        rms = jnp.mean(jnp.square(xf), axis=-1, keepdims=True)
        normed_f = xf * jax.lax.rsqrt(rms + jnp.float32(_EPS))
        normed_f = normed_f * nw_ref[...].astype(jnp.float32)
        normed = normed_f.astype(jnp.bfloat16)
        normedT_ref[...] = normed.T
        gl = jax.lax.dot_general(
            normed, gw_ref[...],
            dimension_numbers=(((1,), (1,)), ((), ())),
            preferred_element_type=jnp.float32)
        logits = gl + gb_ref[...].astype(jnp.float32)
        logitsT = logits.T
        row = jax.lax.broadcasted_iota(jnp.int32, logitsT.shape, 0)
        m1 = jnp.max(logitsT, axis=0, keepdims=True)
        i1 = jnp.min(jnp.where(logitsT == m1, row, _NEXP), axis=0,
                     keepdims=True)
        masked = jnp.where(row == i1, -jnp.inf, logitsT)
        m2 = jnp.max(masked, axis=0, keepdims=True)
        i2 = jnp.min(jnp.where(masked == m2, row, _NEXP), axis=0,
                     keepdims=True)
        eb = jnp.exp(m2 - m1)
        denom = 1.0 + eb
        wa = 1.0 / denom
        wb = eb / denom
        c_ref[...] = (jnp.where(row == i1, wa, 0.0)
                      + jnp.where(row == i2, wb, 0.0))
        b1t_ref[...] = b1_ref[...].astype(jnp.float32).T
        acc_ref[...] = jnp.zeros_like(acc_ref)

    normedT = normedT_ref[...]
    lane = jax.lax.broadcasted_iota(jnp.int32, (_INTER, _NEXP), 1)
    bg = b1t_ref[pl.Slice(0, _INTER, 2), :]
    bl = b1t_ref[pl.Slice(1, _INTER, 2), :]
    contrib = 0.0
    for k in range(2):
        e = 2 * g + k
        h_ref[k] = jax.lax.dot_general(
            w1_ref[k], normedT,
            dimension_numbers=(((1,), (0,)), ((), ())),
            preferred_element_type=jnp.float32)
        hg = h_ref[k, pl.Slice(0, _INTER, 2), :]
        hl = h_ref[k, pl.Slice(1, _INTER, 2), :]
        hg = hg + jnp.sum(jnp.where(lane == e, bg, 0.0), axis=1, keepdims=True)
        hl = hl + jnp.sum(jnp.where(lane == e, bl, 0.0), axis=1, keepdims=True)
        hg = jnp.minimum(hg, _LIMIT)
        hl = jnp.clip(hl, -_LIMIT, _LIMIT)
        act = hg * jax.nn.sigmoid(_ALPHA * hg) * (hl + 1.0)
        y = jax.lax.dot_general(
            w2_ref[k], act.astype(jnp.bfloat16),
            dimension_numbers=(((1,), (0,)), ((), ())),
            preferred_element_type=jnp.float32)
        contrib = contrib + c_ref[pl.ds(e, 1), :] * y
    acc_ref[...] += contrib

    @pl.when(g == _NEXP // 2 - 1)
    def _epilogue():
        b2c = jax.lax.dot_general(
            b2_ref[...].astype(jnp.float32), c_ref[...],
            dimension_numbers=(((0,), (0,)), ((), ())),
            precision=jax.lax.Precision.HIGHEST,
            preferred_element_type=jnp.float32)
        xc = x_ref[0].astype(jnp.bfloat16)
        mixed = (acc_ref[...] + b2c).T
        out_ref[0] = xc + mixed.astype(jnp.bfloat16)


def kernel(x, norm_weight, gate_weight, gate_bias, mlp1_weight, mlp1_bias,
           mlp2_weight, mlp2_bias):
    nw = norm_weight.reshape(1, _HID)
    gb = gate_bias.reshape(1, _NEXP)

    grid = (_NEXP // 2,)
    out = pl.pallas_call(
        _moe_kernel,
        grid=grid,
        in_specs=[
            pl.BlockSpec((_B, _S, _HID), lambda e: (0, 0, 0)),
            pl.BlockSpec((1, _HID), lambda e: (0, 0)),
            pl.BlockSpec((_NEXP, _HID), lambda e: (0, 0)),
            pl.BlockSpec((1, _NEXP), lambda e: (0, 0)),
            pl.BlockSpec((2, 2 * _INTER, _HID), lambda g: (g, 0, 0)),
            pl.BlockSpec((_NEXP, 2 * _INTER), lambda g: (0, 0)),
            pl.BlockSpec((2, _HID, _INTER), lambda g: (g, 0, 0)),
            pl.BlockSpec((_NEXP, _HID), lambda e: (0, 0)),
        ],
        out_specs=pl.BlockSpec((_B, _S, _HID), lambda e: (0, 0, 0)),
        out_shape=jax.ShapeDtypeStruct((_B, _S, _HID), jnp.bfloat16),
        scratch_shapes=[
            pltpu.VMEM((_HID, _S), jnp.bfloat16),
            pltpu.VMEM((_NEXP, _S), jnp.float32),
            pltpu.VMEM((2 * _INTER, _NEXP), jnp.float32),
            pltpu.VMEM((_HID, _S), jnp.float32),
            pltpu.VMEM((2, 2 * _INTER, _S), jnp.float32),
        ],
        compiler_params=pltpu.CompilerParams(
            dimension_semantics=("arbitrary",)),
    )(x, nw, gate_weight, gb, mlp1_weight, mlp1_bias, mlp2_weight, mlp2_bias)
    return out


# speedup vs baseline: 1.0566x; 1.0566x over previous
import jax
import jax.numpy as jnp
from jax.experimental import pallas as pl
from jax.experimental.pallas import tpu as pltpu

_B, _S = 1, 128
_HID = 768
_INTER = 768
_NEXP = 16
_LIMIT = 7.0
_EPS = 1e-05
_ALPHA = 1.702


def _moe_kernel(x_ref, nw_ref, gw_ref, gb_ref,
                w1_ref, b1_ref, w2_ref, b2_ref,
                out_ref,
                normedT_ref, c_ref, b1t_ref, acc_ref, h_ref):
    g = pl.program_id(0)

    @pl.when(g == 0)
    def _prologue():
        xf = x_ref[0].astype(jnp.bfloat16).astype(jnp.float32)
        rms = jnp.mean(jnp.square(xf), axis=-1, keepdims=True)
        normed_f = xf * jax.lax.rsqrt(rms + jnp.float32(_EPS))
        normed_f = normed_f * nw_ref[...].astype(jnp.float32)
        normed = normed_f.astype(jnp.bfloat16)
        normedT_ref[...] = normed.T
        gl = jax.lax.dot_general(
            normed, gw_ref[...],
            dimension_numbers=(((1,), (1,)), ((), ())),
            preferred_element_type=jnp.float32)
        logits = gl + gb_ref[...].astype(jnp.float32)
        logitsT = logits.T
        row = jax.lax.broadcasted_iota(jnp.int32, logitsT.shape, 0)
        m1 = jnp.max(logitsT, axis=0, keepdims=True)
        i1 = jnp.min(jnp.where(logitsT == m1, row, _NEXP), axis=0,
                     keepdims=True)
        masked = jnp.where(row == i1, -jnp.inf, logitsT)
        m2 = jnp.max(masked, axis=0, keepdims=True)
        i2 = jnp.min(jnp.where(masked == m2, row, _NEXP), axis=0,
                     keepdims=True)
        eb = jnp.exp(m2 - m1)
        denom = 1.0 + eb
        wa = 1.0 / denom
        wb = eb / denom
        c_ref[...] = (jnp.where(row == i1, wa, 0.0)
                      + jnp.where(row == i2, wb, 0.0))
        b1t_ref[...] = b1_ref[...].astype(jnp.float32).T
        acc_ref[...] = jnp.zeros_like(acc_ref)

    e = g
    normedT = normedT_ref[...]
    h_ref[...] = jax.lax.dot_general(
        w1_ref[0], normedT,
        dimension_numbers=(((1,), (0,)), ((), ())),
        preferred_element_type=jnp.float32)
    hg = h_ref[pl.Slice(0, _INTER, 2), :]
    hl = h_ref[pl.Slice(1, _INTER, 2), :]
    lane = jax.lax.broadcasted_iota(jnp.int32, (_INTER, _NEXP), 1)
    bg = b1t_ref[pl.Slice(0, _INTER, 2), :]
    bl = b1t_ref[pl.Slice(1, _INTER, 2), :]
    hg = hg + jnp.sum(jnp.where(lane == e, bg, 0.0), axis=1, keepdims=True)
    hl = hl + jnp.sum(jnp.where(lane == e, bl, 0.0), axis=1, keepdims=True)
    hg = jnp.minimum(hg, _LIMIT)
    hl = jnp.clip(hl, -_LIMIT, _LIMIT)
    act = hg * jax.nn.sigmoid(_ALPHA * hg) * (hl + 1.0)
    y = jax.lax.dot_general(
        w2_ref[0], act.astype(jnp.bfloat16),
        dimension_numbers=(((1,), (0,)), ((), ())),
        preferred_element_type=jnp.float32)
    acc_ref[...] += c_ref[pl.ds(e, 1), :] * y

    @pl.when(g == _NEXP - 1)
    def _epilogue():
        b2c = jax.lax.dot_general(
            b2_ref[...].astype(jnp.float32), c_ref[...],
            dimension_numbers=(((0,), (0,)), ((), ())),
            precision=jax.lax.Precision.HIGHEST,
            preferred_element_type=jnp.float32)
        xc = x_ref[0].astype(jnp.bfloat16)
        mixed = (acc_ref[...] + b2c).T
        out_ref[0] = xc + mixed.astype(jnp.bfloat16)


def kernel(x, norm_weight, gate_weight, gate_bias, mlp1_weight, mlp1_bias,
           mlp2_weight, mlp2_bias):
    nw = norm_weight.reshape(1, _HID)
    gb = gate_bias.reshape(1, _NEXP)

    grid = (_NEXP,)
    out = pl.pallas_call(
        _moe_kernel,
        grid=grid,
        in_specs=[
            pl.BlockSpec((_B, _S, _HID), lambda e: (0, 0, 0)),
            pl.BlockSpec((1, _HID), lambda e: (0, 0)),
            pl.BlockSpec((_NEXP, _HID), lambda e: (0, 0)),
            pl.BlockSpec((1, _NEXP), lambda e: (0, 0)),
            pl.BlockSpec((1, 2 * _INTER, _HID), lambda g: (g, 0, 0)),
            pl.BlockSpec((_NEXP, 2 * _INTER), lambda g: (0, 0)),
            pl.BlockSpec((1, _HID, _INTER), lambda g: (g, 0, 0)),
            pl.BlockSpec((_NEXP, _HID), lambda e: (0, 0)),
        ],
        out_specs=pl.BlockSpec((_B, _S, _HID), lambda e: (0, 0, 0)),
        out_shape=jax.ShapeDtypeStruct((_B, _S, _HID), jnp.bfloat16),
        scratch_shapes=[
            pltpu.VMEM((_HID, _S), jnp.bfloat16),
            pltpu.VMEM((_NEXP, _S), jnp.float32),
            pltpu.VMEM((2 * _INTER, _NEXP), jnp.float32),
            pltpu.VMEM((_HID, _S), jnp.float32),
            pltpu.VMEM((2 * _INTER, _S), jnp.float32),
        ],
        compiler_params=pltpu.CompilerParams(
            dimension_semantics=("arbitrary",)),
    )(x, nw, gate_weight, gb, mlp1_weight, mlp1_bias, mlp2_weight, mlp2_bias)
    return out
